# baseline (speedup 1.0000x reference)
import functools

import jax
import jax.numpy as jnp
from jax import lax
from jax.experimental import pallas as pl
from jax.experimental.pallas import tpu as pltpu
from jax.experimental.pallas import tpu_sc as plsc

N = 40020
E = 640320
F = 116
H = 64
NP = 40032
B = 128
GB = 5
EPT = 40960
NG = EPT // (GB * B)
E_PAD = 16 * EPT
E_OVER = E_PAD + GB * B
R2 = E_OVER // B
ROWS_PT = EPT // B
RPT = NP // 16
D1 = 116 * 64
D2 = 116 * 32
G = 345
GP = 352

_SC_MESH = dict(core_axis_name="c", subcore_axis_name="s",
                num_cores=2, num_subcores=16)


def _softplus(v):
    return jnp.where(v > 20.0, v, jnp.log1p(jnp.exp(jnp.minimum(v, 20.0))))


def _mish(v):
    return v * jnp.tanh(_softplus(v))


@functools.lru_cache(maxsize=None)
def _make_sc_agg(nch):
    cpc = nch // 2

    @functools.partial(
        pl.kernel,
        out_type=jax.ShapeDtypeStruct((nch * NP, 32), jnp.float32),
        mesh=plsc.VectorSubcoreMesh(**_SC_MESH),
        scratch_types=[
            pltpu.VMEM((2, GB, B), jnp.int32),
            pltpu.VMEM((2, GB, B), jnp.int32),
            pltpu.VMEM((2, GB, B), jnp.int32),
            pltpu.VMEM((2, GB, B, 32), jnp.float32),
            pltpu.VMEM_SHARED((NP, 32), jnp.float32),
            pltpu.SemaphoreType.DMA,
            pltpu.SemaphoreType.DMA,
            pltpu.SemaphoreType.DMA,
            pltpu.SemaphoreType.DMA,
            pltpu.SemaphoreType.DMA,
            pltpu.SemaphoreType.DMA,
        ],
        compiler_params=pltpu.CompilerParams(use_tc_tiling_on_sc=False),
    )
    def sc_agg(src_hbm, dst_hbm, table_hbm, zeros_hbm, zdrain_hbm, out_hbm,
               src3, dst3, sdst3, rows, acc,
               gsem0, gsem1, ssem0, ssem1, isem0, isem1):
        c = lax.axis_index("c")
        s = lax.axis_index("s")
        row0 = s * RPT
        base_r = s * ROWS_PT
        gsems = (gsem0, gsem1)
        ssems = (ssem0, ssem1)
        isems = (isem0, isem1)

        for ci in range(cpc):
            chunk = c * cpc + ci
            soff = chunk * R2

            def front_half(g, b, goff):
                pltpu.make_async_copy(zdrain_hbm, rows.at[b],
                                      ssems[b]).wait()
                pltpu.make_async_copy(dst_hbm.at[pl.ds(0, GB)],
                                      src3.at[b], isems[b]).wait()
                pltpu.make_async_copy(dst_hbm.at[pl.ds(0, GB)],
                                      dst3.at[b], isems[b]).wait()
                for j in range(GB):
                    for k in range(B // 16):
                        sl = pl.ds(k * 16, 16)
                        sdst3[b, j, sl] = dst3[b, j, sl]
                for j in range(GB):
                    pltpu.async_copy(table_hbm.at[src3.at[b, j]],
                                     rows.at[b, j], gsems[b])

            def back_half(g, b, goff):
                o = 1 - b
                pltpu.make_async_copy(zdrain_hbm, rows.at[o], gsems[o]).wait()
                for j in range(GB):
                    pltpu.async_copy(rows.at[o, j], acc.at[sdst3.at[o, j]],
                                     ssems[o], add=True)
                pltpu.async_copy(src_hbm.at[pl.ds(soff + goff + GB, GB)],
                                 src3.at[o], isems[o])
                pltpu.async_copy(dst_hbm.at[pl.ds(goff + GB, GB)],
                                 dst3.at[o], isems[o])

            pltpu.sync_copy(zeros_hbm.at[pl.ds(row0, RPT)],
                            acc.at[pl.ds(row0, RPT)])
            plsc.subcore_barrier()

            pltpu.async_copy(zdrain_hbm, rows.at[0], ssems[0])
            pltpu.async_copy(zdrain_hbm, rows.at[1], ssems[1])
            pltpu.async_copy(src_hbm.at[pl.ds(soff + base_r, GB)],
                             src3.at[0], isems[0])
            pltpu.async_copy(dst_hbm.at[pl.ds(base_r, GB)],
                             dst3.at[0], isems[0])
            front_half(0, 0, base_r)
            pltpu.async_copy(src_hbm.at[pl.ds(soff + base_r + GB, GB)],
                             src3.at[1], isems[1])
            pltpu.async_copy(dst_hbm.at[pl.ds(base_r + GB, GB)],
                             dst3.at[1], isems[1])

            def pair(h, carry):
                goff1 = base_r + (2 * h + 1) * GB
                front_half(0, 1, goff1)
                back_half(0, 1, goff1)
                goff0 = goff1 + GB
                front_half(0, 0, goff0)
                back_half(0, 0, goff0)
                return carry

            lax.fori_loop(0, NG // 2 - 1, pair, 0)

            goff_l = base_r + (NG - 1) * GB
            front_half(0, 1, goff_l)
            back_half(0, 1, goff_l)
            pltpu.make_async_copy(zdrain_hbm, rows.at[1], gsems[1]).wait()
            for j in range(GB):
                pltpu.async_copy(rows.at[1, j], acc.at[sdst3.at[1, j]],
                                 ssems[1], add=True)
            pltpu.make_async_copy(zdrain_hbm, rows.at[0], ssems[0]).wait()
            pltpu.make_async_copy(zdrain_hbm, rows.at[1], ssems[1]).wait()
            pltpu.make_async_copy(dst_hbm.at[pl.ds(0, GB)],
                                  src3.at[0], isems[0]).wait()
            pltpu.make_async_copy(dst_hbm.at[pl.ds(0, GB)],
                                  dst3.at[0], isems[0]).wait()

            plsc.subcore_barrier()
            pltpu.sync_copy(acc.at[pl.ds(row0, RPT)],
                            out_hbm.at[pl.ds(chunk * NP + row0, RPT)])

    return sc_agg


_BN = 288


def _tc1_body(agg_ref, x_ref, wl1_ref, wr1_ref, bl1_ref, w2_ref, b2_ref,
              out_ref):
    a = agg_ref[...]
    deg = a[:, 116:117]
    dn = 1.0 / jnp.maximum(deg, 1.0)
    acc = jnp.dot(a * dn, wl1_ref[...], preferred_element_type=jnp.float32)
    acc = acc + jnp.dot(x_ref[...], wr1_ref[...],
                        preferred_element_type=jnp.float32)
    h1 = _mish(acc + bl1_ref[...])
    out_ref[...] = jnp.dot(h1, w2_ref[...],
                           preferred_element_type=jnp.float32) + b2_ref[...]


def _tc2_body(a2_ref, ps_ref, c3_ref, out_ref):
    deg = c3_ref[:, 20:21]
    dn = 1.0 / jnp.maximum(deg, 1.0)
    out_ref[...] = _mish(a2_ref[...] * dn + ps_ref[:, 64:128])


def _tc3_body(x_ref, w_ref, b_ref, out_ref):
    out_ref[...] = jnp.dot(x_ref[...], w_ref[...],
                           preferred_element_type=jnp.float32) + b_ref[...]


def _tc4_body(h_ref, gamma_ref, beta_ref, w_ref, b_ref, out_ref):
    h = h_ref[...]
    mu = jnp.mean(h, axis=-1, keepdims=True)
    var = jnp.mean((h - mu) ** 2, axis=-1, keepdims=True)
    hn = (h - mu) * lax.rsqrt(var + 1e-5)
    hn = hn * gamma_ref[...] + beta_ref[...]
    hm = _mish(hn)
    out_ref[...] = jnp.dot(hm, w_ref[...],
                           preferred_element_type=jnp.float32) + b_ref[...]


def _chunked(mat, nch):
    return mat.reshape(NP, nch, 32).transpose(1, 0, 2).reshape(nch * NP, 32)


def _unchunk(mat, nch):
    return mat.reshape(nch, NP, 32).transpose(1, 0, 2).reshape(NP, nch * 32)


def kernel(x, Wl1, bl1, Wr1, Wl2, bl2, Wr2, W1, b1, gamma, beta, W2, b2,
           edge_index):
    f32 = jnp.float32
    src = edge_index[0].astype(jnp.int32)
    dst = edge_index[1].astype(jnp.int32)
    pad_e = jnp.full((E_OVER - E,), N, jnp.int32)
    src_p = jnp.concatenate([src, pad_e]).reshape(R2, B)
    dst_p = jnp.concatenate([dst, pad_e]).reshape(R2, B)
    off4 = (jnp.arange(4, dtype=jnp.int32) * NP)[:, None, None]
    src_all4 = (src_p[None] + off4).reshape(4 * R2, B)
    src_all2 = src_all4[: 2 * R2]
    zdrain = jnp.zeros((GB, B, 32), f32)

    ones_col = jnp.ones((N, 1), f32)
    x1 = jnp.zeros((NP, 128), f32)
    x1 = x1.at[:N, :116].set(x.astype(f32))
    x1 = x1.at[:N, 116:117].set(ones_col)
    xc = _chunked(x1, 4)

    zeros32 = jnp.zeros((NP, 32), f32)

    agg1c = _make_sc_agg(4)(src_all4, dst_p, xc, zeros32, zdrain)
    agg1n = _unchunk(agg1c, 4)

    wl1 = jnp.zeros((128, 128), f32).at[:116, :116].set(Wl1.T)
    wr1 = jnp.zeros((128, 128), f32).at[:116, :116].set(Wr1.T)
    bl1p = jnp.zeros((1, 128), f32).at[0, :116].set(bl1)
    w2cat = jnp.zeros((128, 128), f32)
    w2cat = w2cat.at[:116, :64].set(Wl2.T).at[:116, 64:128].set(Wr2.T)
    b2cat = jnp.zeros((1, 128), f32).at[0, 64:128].set(bl2)

    nblk = NP // _BN
    row_spec = pl.BlockSpec((_BN, 128), lambda i: (i, 0))
    w_spec = pl.BlockSpec((128, 128), lambda i: (0, 0))
    bias_spec = pl.BlockSpec((1, 128), lambda i: (0, 0))
    ps = pl.pallas_call(
        _tc1_body,
        grid=(nblk,),
        in_specs=[row_spec, row_spec, w_spec, w_spec, bias_spec, w_spec,
                  bias_spec],
        out_specs=row_spec,
        out_shape=jax.ShapeDtypeStruct((NP, 128), f32),
    )(agg1n, x1, wl1, wr1, bl1p, w2cat, b2cat)

    p2c = _chunked(ps[:, :64], 2)
    agg2c = _make_sc_agg(2)(src_all2, dst_p, p2c, zeros32, zdrain)
    agg2n = _unchunk(agg2c, 2)

    a2_spec = pl.BlockSpec((_BN, 64), lambda i: (i, 0))
    c3_spec = pl.BlockSpec((_BN, 32), lambda i: (i, 0))
    h2 = pl.pallas_call(
        _tc2_body,
        grid=(nblk,),
        in_specs=[a2_spec, row_spec, c3_spec],
        out_specs=a2_spec,
        out_shape=jax.ShapeDtypeStruct((NP, 64), f32),
    )(agg2n, ps, lax.slice_in_dim(agg1c, 3 * NP, 4 * NP, axis=0))

    x3 = h2[:N].reshape(G, D1)
    x3 = jnp.pad(x3, ((0, GP - G), (0, 0)))
    w1t = W1.T
    b1r = b1.reshape(1, D2)
    hmid = pl.pallas_call(
        _tc3_body,
        grid=(D2 // 128,),
        in_specs=[
            pl.BlockSpec((GP, D1), lambda j: (0, 0)),
            pl.BlockSpec((D1, 128), lambda j: (0, j)),
            pl.BlockSpec((1, 128), lambda j: (0, j)),
        ],
        out_specs=pl.BlockSpec((GP, 128), lambda j: (0, j)),
        out_shape=jax.ShapeDtypeStruct((GP, D2), f32),
    )(x3, w1t, b1r)

    w2t = jnp.zeros((D2, 128), f32).at[:, :2].set(W2.T)
    b2r = jnp.zeros((1, 128), f32).at[0, :2].set(b2)
    out128 = pl.pallas_call(
        _tc4_body,
        in_specs=[
            pl.BlockSpec((GP, D2), lambda: (0, 0)),
            pl.BlockSpec((1, D2), lambda: (0, 0)),
            pl.BlockSpec((1, D2), lambda: (0, 0)),
            pl.BlockSpec((D2, 128), lambda: (0, 0)),
            pl.BlockSpec((1, 128), lambda: (0, 0)),
        ],
        out_specs=pl.BlockSpec((GP, 128), lambda: (0, 0)),
        out_shape=jax.ShapeDtypeStruct((GP, 128), f32),
    )(hmid, gamma.reshape(1, D2), beta.reshape(1, D2), w2t, b2r)

    return out128[:G, :2]


# speedup vs baseline: 1.0605x; 1.0605x over previous
import functools

import jax
import jax.numpy as jnp
from jax import lax
from jax.experimental import pallas as pl
from jax.experimental.pallas import tpu as pltpu
from jax.experimental.pallas import tpu_sc as plsc

N = 40020
E = 640320
F = 116
H = 64
NP = 40032
B = 128
GB = 5
EPT = 40960
NG = EPT // (GB * B)
E_PAD = 16 * EPT
E_OVER = E_PAD + GB * B
R2 = E_OVER // B
ROWS_PT = EPT // B
RPT = NP // 16
D1 = 116 * 64
D2 = 116 * 32
D2P = 3840
G = 345
GP = 352

_SC_MESH = dict(core_axis_name="c", subcore_axis_name="s",
                num_cores=2, num_subcores=16)


def _softplus(v):
    return jnp.where(v > 20.0, v, jnp.log1p(jnp.exp(jnp.minimum(v, 20.0))))


def _mish(v):
    return v * jnp.tanh(_softplus(v))


@functools.lru_cache(maxsize=None)
def _make_sc_agg(nch):
    cpc = nch // 2

    @functools.partial(
        pl.kernel,
        out_type=jax.ShapeDtypeStruct((NP, nch * 32), jnp.float32),
        mesh=plsc.VectorSubcoreMesh(**_SC_MESH),
        scratch_types=[
            pltpu.VMEM((2, GB, B), jnp.int32),
            pltpu.VMEM((2, GB, B), jnp.int32),
            pltpu.VMEM((2, GB, B), jnp.int32),
            pltpu.VMEM((2, GB, B, 32), jnp.float32),
            pltpu.VMEM_SHARED((NP, 32), jnp.float32),
            pltpu.SemaphoreType.DMA,
            pltpu.SemaphoreType.DMA,
            pltpu.SemaphoreType.DMA,
            pltpu.SemaphoreType.DMA,
            pltpu.SemaphoreType.DMA,
            pltpu.SemaphoreType.DMA,
        ],
        compiler_params=pltpu.CompilerParams(use_tc_tiling_on_sc=False),
    )
    def sc_agg(src_hbm, dst_hbm, table_hbm, zeros_hbm, zdrain_hbm, out_hbm,
               src3, dst3, sdst3, rows, acc,
               gsem0, gsem1, ssem0, ssem1, isem0, isem1):
        c = lax.axis_index("c")
        s = lax.axis_index("s")
        row0 = s * RPT
        base_r = s * ROWS_PT
        gsems = (gsem0, gsem1)
        ssems = (ssem0, ssem1)
        isems = (isem0, isem1)

        for ci in range(cpc):
            chunk = c * cpc + ci
            soff = chunk * R2

            def front_half(g, b, goff):
                pltpu.make_async_copy(zdrain_hbm, rows.at[b],
                                      ssems[b]).wait()
                pltpu.make_async_copy(dst_hbm.at[pl.ds(0, GB)],
                                      src3.at[b], isems[b]).wait()
                pltpu.make_async_copy(dst_hbm.at[pl.ds(0, GB)],
                                      dst3.at[b], isems[b]).wait()
                for j in range(GB):
                    for k in range(B // 16):
                        sl = pl.ds(k * 16, 16)
                        sdst3[b, j, sl] = dst3[b, j, sl]
                for j in range(GB):
                    pltpu.async_copy(table_hbm.at[src3.at[b, j]],
                                     rows.at[b, j], gsems[b])

            def back_half(g, b, goff):
                o = 1 - b
                pltpu.make_async_copy(zdrain_hbm, rows.at[o], gsems[o]).wait()
                for j in range(GB):
                    pltpu.async_copy(rows.at[o, j], acc.at[sdst3.at[o, j]],
                                     ssems[o], add=True)
                pltpu.async_copy(src_hbm.at[pl.ds(soff + goff + GB, GB)],
                                 src3.at[o], isems[o])
                pltpu.async_copy(dst_hbm.at[pl.ds(goff + GB, GB)],
                                 dst3.at[o], isems[o])

            pltpu.sync_copy(zeros_hbm.at[pl.ds(row0, RPT)],
                            acc.at[pl.ds(row0, RPT)])
            plsc.subcore_barrier()

            pltpu.async_copy(zdrain_hbm, rows.at[0], ssems[0])
            pltpu.async_copy(zdrain_hbm, rows.at[1], ssems[1])
            pltpu.async_copy(src_hbm.at[pl.ds(soff + base_r, GB)],
                             src3.at[0], isems[0])
            pltpu.async_copy(dst_hbm.at[pl.ds(base_r, GB)],
                             dst3.at[0], isems[0])
            front_half(0, 0, base_r)
            pltpu.async_copy(src_hbm.at[pl.ds(soff + base_r + GB, GB)],
                             src3.at[1], isems[1])
            pltpu.async_copy(dst_hbm.at[pl.ds(base_r + GB, GB)],
                             dst3.at[1], isems[1])

            def pair(h, carry):
                goff1 = base_r + (2 * h + 1) * GB
                front_half(0, 1, goff1)
                back_half(0, 1, goff1)
                goff0 = goff1 + GB
                front_half(0, 0, goff0)
                back_half(0, 0, goff0)
                return carry

            lax.fori_loop(0, NG // 2 - 1, pair, 0)

            goff_l = base_r + (NG - 1) * GB
            front_half(0, 1, goff_l)
            back_half(0, 1, goff_l)
            pltpu.make_async_copy(zdrain_hbm, rows.at[1], gsems[1]).wait()
            for j in range(GB):
                pltpu.async_copy(rows.at[1, j], acc.at[sdst3.at[1, j]],
                                 ssems[1], add=True)
            pltpu.make_async_copy(zdrain_hbm, rows.at[0], ssems[0]).wait()
            pltpu.make_async_copy(zdrain_hbm, rows.at[1], ssems[1]).wait()
            pltpu.make_async_copy(dst_hbm.at[pl.ds(0, GB)],
                                  src3.at[0], isems[0]).wait()
            pltpu.make_async_copy(dst_hbm.at[pl.ds(0, GB)],
                                  dst3.at[0], isems[0]).wait()

            plsc.subcore_barrier()
            pltpu.sync_copy(acc.at[pl.ds(row0, RPT)],
                            out_hbm.at[pl.ds(row0, RPT),
                                       pl.ds(chunk * 32, 32)])

    return sc_agg


_BN = 288


def _tc1_body(agg_ref, x_ref, wl1_ref, wr1_ref, bl1_ref, w2_ref, b2_ref,
              p2_ref, s2_ref):
    a = agg_ref[...]
    deg = a[:, 116:117]
    dn = 1.0 / jnp.maximum(deg, 1.0)
    acc = jnp.dot(a * dn, wl1_ref[...], preferred_element_type=jnp.float32)
    acc = acc + jnp.dot(x_ref[...], wr1_ref[...],
                        preferred_element_type=jnp.float32)
    h1 = _mish(acc + bl1_ref[...])
    ps = jnp.dot(h1, w2_ref[...],
                 preferred_element_type=jnp.float32) + b2_ref[...]
    p2_ref[...] = ps[:, :64]
    s2_ref[...] = ps[:, 64:128]


def _tc2_body(a2_ref, s2_ref, agg1_ref, out_ref):
    deg = agg1_ref[:, 116:117]
    dn = 1.0 / jnp.maximum(deg, 1.0)
    out_ref[...] = _mish(a2_ref[...] * dn + s2_ref[...])


def _tc3_body(x_ref, w_ref, b_ref, out_ref):
    out_ref[...] = jnp.dot(x_ref[...], w_ref[...],
                           preferred_element_type=jnp.float32) + b_ref[...]


def _tc4_body(h_ref, gamma_ref, beta_ref, w_ref, b_ref, out_ref):
    h = h_ref[...]
    mask = (lax.broadcasted_iota(jnp.int32, (1, D2P), 1) < D2).astype(h.dtype)
    mu = jnp.sum(h, axis=-1, keepdims=True) * (1.0 / D2)
    diff = (h - mu) * mask
    var = jnp.sum(diff * diff, axis=-1, keepdims=True) * (1.0 / D2)
    hn = diff * lax.rsqrt(var + 1e-5)
    hn = hn * gamma_ref[...] + beta_ref[...]
    hm = _mish(hn)
    out_ref[...] = jnp.dot(hm, w_ref[...],
                           preferred_element_type=jnp.float32) + b_ref[...]


def kernel(x, Wl1, bl1, Wr1, Wl2, bl2, Wr2, W1, b1, gamma, beta, W2, b2,
           edge_index):
    f32 = jnp.float32
    src = edge_index[0].astype(jnp.int32)
    dst = edge_index[1].astype(jnp.int32)
    pad_e = jnp.full((E_OVER - E,), N, jnp.int32)
    src_p = jnp.concatenate([src, pad_e]).reshape(R2, B)
    dst_p = jnp.concatenate([dst, pad_e]).reshape(R2, B)
    c4 = jnp.arange(4, dtype=jnp.int32)[:, None, None]
    src_all4 = (src_p[None] * 4 + c4).reshape(4 * R2, B)
    c2 = jnp.arange(2, dtype=jnp.int32)[:, None, None]
    src_all2 = (src_p[None] * 2 + c2).reshape(2 * R2, B)
    zdrain = jnp.zeros((GB, B, 32), f32)

    ones_col = jnp.ones((N, 1), f32)
    x1 = jnp.zeros((NP, 128), f32)
    x1 = x1.at[:N, :116].set(x.astype(f32))
    x1 = x1.at[:N, 116:117].set(ones_col)
    xc = x1.reshape(4 * NP, 32)

    zeros32 = jnp.zeros((NP, 32), f32)

    agg1n = _make_sc_agg(4)(src_all4, dst_p, xc, zeros32, zdrain)

    wl1 = jnp.zeros((128, 128), f32).at[:116, :116].set(Wl1.T)
    wr1 = jnp.zeros((128, 128), f32).at[:116, :116].set(Wr1.T)
    bl1p = jnp.zeros((1, 128), f32).at[0, :116].set(bl1)
    w2cat = jnp.zeros((128, 128), f32)
    w2cat = w2cat.at[:116, :64].set(Wl2.T).at[:116, 64:128].set(Wr2.T)
    b2cat = jnp.zeros((1, 128), f32).at[0, 64:128].set(bl2)

    nblk = NP // _BN
    row_spec = pl.BlockSpec((_BN, 128), lambda i: (i, 0))
    half_spec = pl.BlockSpec((_BN, 64), lambda i: (i, 0))
    w_spec = pl.BlockSpec((128, 128), lambda i: (0, 0))
    bias_spec = pl.BlockSpec((1, 128), lambda i: (0, 0))
    p2, s2 = pl.pallas_call(
        _tc1_body,
        grid=(nblk,),
        in_specs=[row_spec, row_spec, w_spec, w_spec, bias_spec, w_spec,
                  bias_spec],
        out_specs=[half_spec, half_spec],
        out_shape=[jax.ShapeDtypeStruct((NP, 64), f32),
                   jax.ShapeDtypeStruct((NP, 64), f32)],
    )(agg1n, x1, wl1, wr1, bl1p, w2cat, b2cat)

    p2c = p2.reshape(2 * NP, 32)
    agg2n = _make_sc_agg(2)(src_all2, dst_p, p2c, zeros32, zdrain)

    h2 = pl.pallas_call(
        _tc2_body,
        grid=(nblk,),
        in_specs=[half_spec, half_spec, row_spec],
        out_specs=half_spec,
        out_shape=jax.ShapeDtypeStruct((NP, 64), f32),
    )(agg2n, s2, agg1n)

    x3 = h2[:N].reshape(G, D1)
    x3 = jnp.pad(x3, ((0, GP - G), (0, 0)))
    w1t = jnp.pad(W1.T, ((0, 0), (0, D2P - D2)))
    b1r = jnp.pad(b1, (0, D2P - D2)).reshape(1, D2P)
    hmid = pl.pallas_call(
        _tc3_body,
        grid=(D2P // 256,),
        in_specs=[
            pl.BlockSpec((GP, D1), lambda j: (0, 0)),
            pl.BlockSpec((D1, 256), lambda j: (0, j)),
            pl.BlockSpec((1, 256), lambda j: (0, j)),
        ],
        out_specs=pl.BlockSpec((GP, 256), lambda j: (0, j)),
        out_shape=jax.ShapeDtypeStruct((GP, D2P), f32),
    )(x3, w1t, b1r)

    w2t = jnp.zeros((D2P, 128), f32).at[:D2, :2].set(W2.T)
    b2r = jnp.zeros((1, 128), f32).at[0, :2].set(b2)
    gammap = jnp.pad(gamma, (0, D2P - D2)).reshape(1, D2P)
    betap = jnp.pad(beta, (0, D2P - D2)).reshape(1, D2P)
    out128 = pl.pallas_call(
        _tc4_body,
        in_specs=[
            pl.BlockSpec((GP, D2P), lambda: (0, 0)),
            pl.BlockSpec((1, D2P), lambda: (0, 0)),
            pl.BlockSpec((1, D2P), lambda: (0, 0)),
            pl.BlockSpec((D2P, 128), lambda: (0, 0)),
            pl.BlockSpec((1, 128), lambda: (0, 0)),
        ],
        out_specs=pl.BlockSpec((GP, 128), lambda: (0, 0)),
        out_shape=jax.ShapeDtypeStruct((GP, 128), f32),
    )(hmid, gammap, betap, w2t, b2r)

    return out128[:G, :2]


# speedup vs baseline: 1.0674x; 1.0066x over previous
import functools

import jax
import jax.numpy as jnp
from jax import lax
from jax.experimental import pallas as pl
from jax.experimental.pallas import tpu as pltpu
from jax.experimental.pallas import tpu_sc as plsc

N = 40020
E = 640320
F = 116
H = 64
NP = 40032
B = 128
GB = 5
EPT = 40960
NG = EPT // (GB * B)
E_PAD = 16 * EPT
E_OVER = E_PAD + GB * B
R2 = E_OVER // B
ROWS_PT = EPT // B
RPT = NP // 16
D1 = 116 * 64
D2 = 116 * 32
D2P = 3840
G = 345
GP = 352

_SC_MESH = dict(core_axis_name="c", subcore_axis_name="s",
                num_cores=2, num_subcores=16)


def _softplus(v):
    return jnp.where(v > 20.0, v, jnp.log1p(jnp.exp(jnp.minimum(v, 20.0))))


def _mish(v):
    return v * jnp.tanh(_softplus(v))


@functools.lru_cache(maxsize=None)
def _make_sc_agg(nch):
    cpc = nch // 2

    @functools.partial(
        pl.kernel,
        out_type=jax.ShapeDtypeStruct((nch * NP, 32), jnp.float32),
        mesh=plsc.VectorSubcoreMesh(**_SC_MESH),
        scratch_types=[
            pltpu.VMEM((2, GB, B), jnp.int32),
            pltpu.VMEM((2, GB, B), jnp.int32),
            pltpu.VMEM((2, GB, B), jnp.int32),
            pltpu.VMEM((2, GB, B, 32), jnp.float32),
            pltpu.VMEM_SHARED((NP, 32), jnp.float32),
            pltpu.SemaphoreType.DMA,
            pltpu.SemaphoreType.DMA,
            pltpu.SemaphoreType.DMA,
            pltpu.SemaphoreType.DMA,
            pltpu.SemaphoreType.DMA,
            pltpu.SemaphoreType.DMA,
        ],
        compiler_params=pltpu.CompilerParams(use_tc_tiling_on_sc=False),
    )
    def sc_agg(src_hbm, dst_hbm, table_hbm, zeros_hbm, zdrain_hbm, out_hbm,
               src3, dst3, sdst3, rows, acc,
               gsem0, gsem1, ssem0, ssem1, isem0, isem1):
        c = lax.axis_index("c")
        s = lax.axis_index("s")
        tbl = table_hbm
        row0 = s * RPT
        base_r = s * ROWS_PT
        gsems = (gsem0, gsem1)
        ssems = (ssem0, ssem1)
        isems = (isem0, isem1)

        for ci in range(cpc):
            chunk = c * cpc + ci
            soff = chunk * R2

            def front_half(g, b, goff):
                pltpu.make_async_copy(zdrain_hbm, rows.at[b],
                                      ssems[b]).wait()
                pltpu.make_async_copy(dst_hbm.at[pl.ds(0, GB)],
                                      src3.at[b], isems[b]).wait()
                pltpu.make_async_copy(dst_hbm.at[pl.ds(0, GB)],
                                      dst3.at[b], isems[b]).wait()
                for j in range(GB):
                    for k in range(B // 16):
                        sl = pl.ds(k * 16, 16)
                        sdst3[b, j, sl] = dst3[b, j, sl]
                for j in range(GB):
                    pltpu.async_copy(tbl.at[src3.at[b, j]],
                                     rows.at[b, j], gsems[b])

            def back_half(g, b, goff):
                o = 1 - b
                pltpu.make_async_copy(zdrain_hbm, rows.at[o], gsems[o]).wait()
                for j in range(GB):
                    pltpu.async_copy(rows.at[o, j], acc.at[sdst3.at[o, j]],
                                     ssems[o], add=True)
                pltpu.async_copy(src_hbm.at[pl.ds(soff + goff + GB, GB)],
                                 src3.at[o], isems[o])
                pltpu.async_copy(dst_hbm.at[pl.ds(goff + GB, GB)],
                                 dst3.at[o], isems[o])

            pltpu.sync_copy(zeros_hbm.at[pl.ds(row0, RPT)],
                            acc.at[pl.ds(row0, RPT)])
            plsc.subcore_barrier()

            pltpu.async_copy(zdrain_hbm, rows.at[0], ssems[0])
            pltpu.async_copy(zdrain_hbm, rows.at[1], ssems[1])
            pltpu.async_copy(src_hbm.at[pl.ds(soff + base_r, GB)],
                             src3.at[0], isems[0])
            pltpu.async_copy(dst_hbm.at[pl.ds(base_r, GB)],
                             dst3.at[0], isems[0])
            front_half(0, 0, base_r)
            pltpu.async_copy(src_hbm.at[pl.ds(soff + base_r + GB, GB)],
                             src3.at[1], isems[1])
            pltpu.async_copy(dst_hbm.at[pl.ds(base_r + GB, GB)],
                             dst3.at[1], isems[1])

            def pair(h, carry):
                goff1 = base_r + (2 * h + 1) * GB
                front_half(0, 1, goff1)
                back_half(0, 1, goff1)
                goff0 = goff1 + GB
                front_half(0, 0, goff0)
                back_half(0, 0, goff0)
                return carry

            lax.fori_loop(0, NG // 2 - 1, pair, 0)

            goff_l = base_r + (NG - 1) * GB
            front_half(0, 1, goff_l)
            back_half(0, 1, goff_l)
            pltpu.make_async_copy(zdrain_hbm, rows.at[1], gsems[1]).wait()
            for j in range(GB):
                pltpu.async_copy(rows.at[1, j], acc.at[sdst3.at[1, j]],
                                 ssems[1], add=True)
            pltpu.make_async_copy(zdrain_hbm, rows.at[0], ssems[0]).wait()
            pltpu.make_async_copy(zdrain_hbm, rows.at[1], ssems[1]).wait()
            pltpu.make_async_copy(dst_hbm.at[pl.ds(0, GB)],
                                  src3.at[0], isems[0]).wait()
            pltpu.make_async_copy(dst_hbm.at[pl.ds(0, GB)],
                                  dst3.at[0], isems[0]).wait()

            plsc.subcore_barrier()
            pltpu.sync_copy(acc.at[pl.ds(row0, RPT)],
                            out_hbm.at[pl.ds(chunk * NP + row0, RPT)])

    return sc_agg


_BN = 288


def _unshuffle(v, nch):
    bn4 = v.shape[1]
    rows = []
    for k in range(4):
        rows.append(jnp.concatenate(
            [v[c, :, 32 * k:32 * k + 32] for c in range(nch)], axis=1))
    return jnp.stack(rows, axis=1).reshape(bn4 * 4, nch * 32)


def _tc1_body(agg_ref, x_ref, wl1_ref, wr1_ref, bl1_ref, w2_ref, b2_ref,
              p2_ref, s2_ref, dn_ref):
    a = _unshuffle(agg_ref[...], 4)
    deg = a[:, 116:117]
    dn = 1.0 / jnp.maximum(deg, 1.0)
    acc = jnp.dot(a * dn, wl1_ref[...], preferred_element_type=jnp.float32)
    acc = acc + jnp.dot(x_ref[...], wr1_ref[...],
                        preferred_element_type=jnp.float32)
    h1 = _mish(acc + bl1_ref[...])
    ps = jnp.dot(h1, w2_ref[...],
                 preferred_element_type=jnp.float32) + b2_ref[...]
    bn = ps.shape[0]
    p2_ref[...] = ps[:, :64]
    s2_ref[...] = ps[:, 64:128]
    dn_ref[...] = jnp.broadcast_to(dn, (bn, 8))


def _tc2_body(a2_ref, s2_ref, dn_ref, out_ref):
    a2 = _unshuffle(a2_ref[...], 2)
    dn = dn_ref[:, 0:1]
    out_ref[...] = _mish(a2 * dn + s2_ref[...])


def _tc3_body(x_ref, w_ref, b_ref, out_ref):
    out_ref[...] = jnp.dot(x_ref[...], w_ref[...],
                           preferred_element_type=jnp.float32) + b_ref[...]


def _tc4_body(h_ref, gamma_ref, beta_ref, w_ref, b_ref, out_ref):
    h = h_ref[...]
    mask = (lax.broadcasted_iota(jnp.int32, (1, D2P), 1) < D2).astype(h.dtype)
    mu = jnp.sum(h, axis=-1, keepdims=True) * (1.0 / D2)
    diff = (h - mu) * mask
    var = jnp.sum(diff * diff, axis=-1, keepdims=True) * (1.0 / D2)
    hn = diff * lax.rsqrt(var + 1e-5)
    hn = hn * gamma_ref[...] + beta_ref[...]
    hm = _mish(hn)
    out_ref[...] = jnp.dot(hm, w_ref[...],
                           preferred_element_type=jnp.float32) + b_ref[...]


def kernel(x, Wl1, bl1, Wr1, Wl2, bl2, Wr2, W1, b1, gamma, beta, W2, b2,
           edge_index):
    f32 = jnp.float32
    src = edge_index[0].astype(jnp.int32)
    dst = edge_index[1].astype(jnp.int32)
    pad_e = jnp.full((E_OVER - E,), N, jnp.int32)
    src_p = jnp.concatenate([src, pad_e]).reshape(R2, B)
    dst_p = jnp.concatenate([dst, pad_e]).reshape(R2, B)
    c4 = jnp.arange(4, dtype=jnp.int32)[:, None, None]
    src_all4 = (src_p[None] * 4 + c4).reshape(4 * R2, B)
    c2 = jnp.arange(2, dtype=jnp.int32)[:, None, None]
    src_all2 = (src_p[None] * 2 + c2).reshape(2 * R2, B)
    zdrain = jnp.zeros((GB, B, 32), f32)

    ones_col = jnp.ones((N, 1), f32)
    x1 = jnp.zeros((NP, 128), f32)
    x1 = x1.at[:N, :116].set(x.astype(f32))
    x1 = x1.at[:N, 116:117].set(ones_col)

    zeros32 = jnp.zeros((NP, 32), f32)

    agg1c = _make_sc_agg(4)(src_all4, dst_p, x1.reshape(4 * NP, 32),
                            zeros32, zdrain)
    agg1v = agg1c.reshape(4, NP // 4, 128)

    wl1 = jnp.zeros((128, 128), f32).at[:116, :116].set(Wl1.T)
    wr1 = jnp.zeros((128, 128), f32).at[:116, :116].set(Wr1.T)
    bl1p = jnp.zeros((1, 128), f32).at[0, :116].set(bl1)
    w2cat = jnp.zeros((128, 128), f32)
    w2cat = w2cat.at[:116, :64].set(Wl2.T).at[:116, 64:128].set(Wr2.T)
    b2cat = jnp.zeros((1, 128), f32).at[0, 64:128].set(bl2)

    nblk = NP // _BN
    bn4 = _BN // 4
    row_spec = pl.BlockSpec((_BN, 128), lambda i: (i, 0))
    half_spec = pl.BlockSpec((_BN, 64), lambda i: (i, 0))
    agg4_spec = pl.BlockSpec((4, bn4, 128), lambda i: (0, i, 0))
    agg2_spec = pl.BlockSpec((2, bn4, 128), lambda i: (0, i, 0))
    w_spec = pl.BlockSpec((128, 128), lambda i: (0, 0))
    bias_spec = pl.BlockSpec((1, 128), lambda i: (0, 0))
    p2, s2, dninv = pl.pallas_call(
        _tc1_body,
        grid=(nblk,),
        in_specs=[agg4_spec, row_spec, w_spec, w_spec, bias_spec, w_spec,
                  bias_spec],
        out_specs=[half_spec,
                   half_spec,
                   pl.BlockSpec((_BN, 8), lambda i: (i, 0))],
        out_shape=[jax.ShapeDtypeStruct((NP, 64), f32),
                   jax.ShapeDtypeStruct((NP, 64), f32),
                   jax.ShapeDtypeStruct((NP, 8), f32)],
    )(agg1v, x1, wl1, wr1, bl1p, w2cat, b2cat)

    agg2c = _make_sc_agg(2)(src_all2, dst_p, p2.reshape(2 * NP, 32),
                            zeros32, zdrain)
    agg2v = agg2c.reshape(2, NP // 4, 128)

    h2 = pl.pallas_call(
        _tc2_body,
        grid=(nblk,),
        in_specs=[agg2_spec, half_spec,
                  pl.BlockSpec((_BN, 8), lambda i: (i, 0))],
        out_specs=half_spec,
        out_shape=jax.ShapeDtypeStruct((NP, 64), f32),
    )(agg2v, s2, dninv)

    x3 = h2[:N].reshape(G, D1)
    x3 = jnp.pad(x3, ((0, GP - G), (0, 0)))
    w1t = jnp.pad(W1.T, ((0, 0), (0, D2P - D2)))
    b1r = jnp.pad(b1, (0, D2P - D2)).reshape(1, D2P)
    hmid = pl.pallas_call(
        _tc3_body,
        grid=(D2P // 256,),
        in_specs=[
            pl.BlockSpec((GP, D1), lambda j: (0, 0)),
            pl.BlockSpec((D1, 256), lambda j: (0, j)),
            pl.BlockSpec((1, 256), lambda j: (0, j)),
        ],
        out_specs=pl.BlockSpec((GP, 256), lambda j: (0, j)),
        out_shape=jax.ShapeDtypeStruct((GP, D2P), f32),
    )(x3, w1t, b1r)

    w2t = jnp.zeros((D2P, 128), f32).at[:D2, :2].set(W2.T)
    b2r = jnp.zeros((1, 128), f32).at[0, :2].set(b2)
    gammap = jnp.pad(gamma, (0, D2P - D2)).reshape(1, D2P)
    betap = jnp.pad(beta, (0, D2P - D2)).reshape(1, D2P)
    out128 = pl.pallas_call(
        _tc4_body,
        in_specs=[
            pl.BlockSpec((GP, D2P), lambda: (0, 0)),
            pl.BlockSpec((1, D2P), lambda: (0, 0)),
            pl.BlockSpec((1, D2P), lambda: (0, 0)),
            pl.BlockSpec((D2P, 128), lambda: (0, 0)),
            pl.BlockSpec((1, 128), lambda: (0, 0)),
        ],
        out_specs=pl.BlockSpec((GP, 128), lambda: (0, 0)),
        out_shape=jax.ShapeDtypeStruct((GP, 128), f32),
    )(hmid, gammap, betap, w2t, b2r)

    return out128[:G, :2]


# speedup vs baseline: 1.1376x; 1.0657x over previous
import functools

import jax
import jax.numpy as jnp
from jax import lax
from jax.experimental import pallas as pl
from jax.experimental.pallas import tpu as pltpu
from jax.experimental.pallas import tpu_sc as plsc

N = 40020
E = 640320
F = 116
H = 64
NP = 40032
B = 128
GB = 5
EPT = 40960
NG = EPT // (GB * B)
E_PAD = 16 * EPT
E_OVER = E_PAD + GB * B
R2 = E_OVER // B
ROWS_PT = EPT // B
RPT = NP // 16
D1 = 116 * 64
D2 = 116 * 32
D2P = 3840
G = 345
GP = 352

_SC_MESH = dict(core_axis_name="c", subcore_axis_name="s",
                num_cores=2, num_subcores=16)


def _softplus(v):
    return jnp.where(v > 20.0, v, jnp.log1p(jnp.exp(jnp.minimum(v, 20.0))))


def _mish(v):
    return v * jnp.tanh(_softplus(v))


@functools.lru_cache(maxsize=None)
def _make_sc_agg(nch):
    cpc = nch // 2

    @functools.partial(
        pl.kernel,
        out_type=jax.ShapeDtypeStruct((nch * NP, 32), jnp.float32),
        mesh=plsc.VectorSubcoreMesh(**_SC_MESH),
        scratch_types=[
            pltpu.VMEM((2, GB, B), jnp.int32),
            pltpu.VMEM((2, GB, B), jnp.int32),
            pltpu.VMEM((2, GB, B), jnp.int32),
            pltpu.VMEM((2, GB, B, 32), jnp.float32),
            pltpu.VMEM_SHARED((NP, 32), jnp.float32),
            pltpu.SemaphoreType.DMA,
            pltpu.SemaphoreType.DMA,
            pltpu.SemaphoreType.DMA,
            pltpu.SemaphoreType.DMA,
            pltpu.SemaphoreType.DMA,
            pltpu.SemaphoreType.DMA,
        ],
        compiler_params=pltpu.CompilerParams(use_tc_tiling_on_sc=False),
    )
    def sc_agg(src_hbm, dst_hbm, table_hbm, zeros_hbm, zdrain_hbm, out_hbm,
               src3, dst3, sdst3, rows, acc,
               gsem0, gsem1, ssem0, ssem1, isem0, isem1):
        c = lax.axis_index("c")
        s = lax.axis_index("s")
        tbl = table_hbm
        row0 = s * RPT
        base_r = s * ROWS_PT
        gsems = (gsem0, gsem1)
        ssems = (ssem0, ssem1)
        isems = (isem0, isem1)

        for ci in range(cpc):
            chunk = c * cpc + ci
            soff = chunk * R2

            def front_half(g, b, goff):
                pltpu.make_async_copy(zdrain_hbm, rows.at[b],
                                      ssems[b]).wait()
                pltpu.make_async_copy(dst_hbm.at[pl.ds(0, GB)],
                                      src3.at[b], isems[b]).wait()
                pltpu.make_async_copy(dst_hbm.at[pl.ds(0, GB)],
                                      dst3.at[b], isems[b]).wait()
                for j in range(GB):
                    for k in range(B // 16):
                        sl = pl.ds(k * 16, 16)
                        sdst3[b, j, sl] = dst3[b, j, sl]
                for j in range(GB):
                    pltpu.async_copy(tbl.at[src3.at[b, j]],
                                     rows.at[b, j], gsems[b])

            def back_half(g, b, goff):
                o = 1 - b
                pltpu.make_async_copy(zdrain_hbm, rows.at[o], gsems[o]).wait()
                for j in range(GB):
                    pltpu.async_copy(rows.at[o, j], acc.at[sdst3.at[o, j]],
                                     ssems[o], add=True)
                pltpu.async_copy(src_hbm.at[pl.ds(soff + goff + GB, GB)],
                                 src3.at[o], isems[o])
                pltpu.async_copy(dst_hbm.at[pl.ds(goff + GB, GB)],
                                 dst3.at[o], isems[o])

            pltpu.sync_copy(zeros_hbm.at[pl.ds(row0, RPT)],
                            acc.at[pl.ds(row0, RPT)])
            plsc.subcore_barrier()

            pltpu.async_copy(zdrain_hbm, rows.at[0], ssems[0])
            pltpu.async_copy(zdrain_hbm, rows.at[1], ssems[1])
            pltpu.async_copy(src_hbm.at[pl.ds(soff + base_r, GB)],
                             src3.at[0], isems[0])
            pltpu.async_copy(dst_hbm.at[pl.ds(base_r, GB)],
                             dst3.at[0], isems[0])
            front_half(0, 0, base_r)
            pltpu.async_copy(src_hbm.at[pl.ds(soff + base_r + GB, GB)],
                             src3.at[1], isems[1])
            pltpu.async_copy(dst_hbm.at[pl.ds(base_r + GB, GB)],
                             dst3.at[1], isems[1])

            def pair(h, carry):
                goff1 = base_r + (2 * h + 1) * GB
                front_half(0, 1, goff1)
                back_half(0, 1, goff1)
                goff0 = goff1 + GB
                front_half(0, 0, goff0)
                back_half(0, 0, goff0)
                return carry

            lax.fori_loop(0, NG // 2 - 1, pair, 0)

            goff_l = base_r + (NG - 1) * GB
            front_half(0, 1, goff_l)
            back_half(0, 1, goff_l)
            pltpu.make_async_copy(zdrain_hbm, rows.at[1], gsems[1]).wait()
            for j in range(GB):
                pltpu.async_copy(rows.at[1, j], acc.at[sdst3.at[1, j]],
                                 ssems[1], add=True)
            pltpu.make_async_copy(zdrain_hbm, rows.at[0], ssems[0]).wait()
            pltpu.make_async_copy(zdrain_hbm, rows.at[1], ssems[1]).wait()
            pltpu.make_async_copy(dst_hbm.at[pl.ds(0, GB)],
                                  src3.at[0], isems[0]).wait()
            pltpu.make_async_copy(dst_hbm.at[pl.ds(0, GB)],
                                  dst3.at[0], isems[0]).wait()

            plsc.subcore_barrier()
            pltpu.sync_copy(acc.at[pl.ds(row0, RPT)],
                            out_hbm.at[pl.ds(chunk * NP + row0, RPT)])

    return sc_agg


_BN = 288


def _unshuffle(v, nch):
    bn4 = v.shape[1]
    rows = []
    for k in range(4):
        rows.append(jnp.concatenate(
            [v[c, :, 32 * k:32 * k + 32] for c in range(nch)], axis=1))
    return jnp.stack(rows, axis=1).reshape(bn4 * 4, nch * 32)


def _tc1_body(agg_ref, x_ref, wl1_ref, wr1_ref, bl1_ref, w2_ref, b2_ref,
              p2_ref, s2_ref, dn_ref):
    a = _unshuffle(agg_ref[...], 4)
    deg = a[:, 116:117]
    dn = 1.0 / jnp.maximum(deg, 1.0)
    acc = jnp.dot(a * dn, wl1_ref[...], preferred_element_type=jnp.float32)
    acc = acc + jnp.dot(x_ref[...], wr1_ref[...],
                        preferred_element_type=jnp.float32)
    h1 = _mish(acc + bl1_ref[...])
    ps = jnp.dot(h1, w2_ref[...],
                 preferred_element_type=jnp.float32) + b2_ref[...]
    bn = ps.shape[0]
    p2_ref[...] = ps[:, :64]
    s2_ref[...] = ps[:, 64:128]
    dn_ref[...] = jnp.broadcast_to(dn, (bn, 8))


def _tc2_body(a2_ref, s2_ref, dn_ref, out_ref):
    a2 = _unshuffle(a2_ref[...], 2)
    dn = dn_ref[:, 0:1]
    out_ref[...] = _mish(a2 * dn + s2_ref[...])


def _tc3_body(x_ref, w_ref, b_ref, out_ref):
    out_ref[...] = jnp.dot(x_ref[...], w_ref[...],
                           preferred_element_type=jnp.float32) + b_ref[...]


def _tc4_body(h_ref, gamma_ref, beta_ref, w_ref, b_ref, out_ref):
    h = h_ref[...]
    mask = (lax.broadcasted_iota(jnp.int32, (1, D2P), 1) < D2).astype(h.dtype)
    mu = jnp.sum(h, axis=-1, keepdims=True) * (1.0 / D2)
    diff = (h - mu) * mask
    var = jnp.sum(diff * diff, axis=-1, keepdims=True) * (1.0 / D2)
    hn = diff * lax.rsqrt(var + 1e-5)
    hn = hn * gamma_ref[...] + beta_ref[...]
    hm = _mish(hn)
    out_ref[...] = jnp.dot(hm, w_ref[...],
                           preferred_element_type=jnp.float32) + b_ref[...]


def kernel(x, Wl1, bl1, Wr1, Wl2, bl2, Wr2, W1, b1, gamma, beta, W2, b2,
           edge_index):
    f32 = jnp.float32
    src = edge_index[0].astype(jnp.int32)
    dst = edge_index[1].astype(jnp.int32)
    pad_e = jnp.full((E_OVER - E,), N, jnp.int32)
    src_p = jnp.concatenate([src, pad_e]).reshape(R2, B)
    dst_p = jnp.concatenate([dst, pad_e]).reshape(R2, B)
    off4 = (jnp.arange(4, dtype=jnp.int32) * NP)[:, None, None]
    src_all4 = (src_p[None] + off4).reshape(4 * R2, B)
    src_all2 = src_all4[: 2 * R2]
    zdrain = jnp.zeros((GB, B, 32), f32)

    ones_col = jnp.ones((N, 1), f32)
    x1 = jnp.zeros((NP, 128), f32)
    x1 = x1.at[:N, :116].set(x.astype(f32))
    x1 = x1.at[:N, 116:117].set(ones_col)

    zeros32 = jnp.zeros((NP, 32), f32)

    xc = x1.reshape(NP, 4, 32).transpose(1, 0, 2).reshape(4 * NP, 32)
    agg1c = _make_sc_agg(4)(src_all4, dst_p, xc, zeros32, zdrain)
    agg1v = agg1c.reshape(4, NP // 4, 128)

    wl1 = jnp.zeros((128, 128), f32).at[:116, :116].set(Wl1.T)
    wr1 = jnp.zeros((128, 128), f32).at[:116, :116].set(Wr1.T)
    bl1p = jnp.zeros((1, 128), f32).at[0, :116].set(bl1)
    w2cat = jnp.zeros((128, 128), f32)
    w2cat = w2cat.at[:116, :64].set(Wl2.T).at[:116, 64:128].set(Wr2.T)
    b2cat = jnp.zeros((1, 128), f32).at[0, 64:128].set(bl2)

    nblk = NP // _BN
    bn4 = _BN // 4
    row_spec = pl.BlockSpec((_BN, 128), lambda i: (i, 0))
    half_spec = pl.BlockSpec((_BN, 64), lambda i: (i, 0))
    agg4_spec = pl.BlockSpec((4, bn4, 128), lambda i: (0, i, 0))
    agg2_spec = pl.BlockSpec((2, bn4, 128), lambda i: (0, i, 0))
    w_spec = pl.BlockSpec((128, 128), lambda i: (0, 0))
    bias_spec = pl.BlockSpec((1, 128), lambda i: (0, 0))
    p2, s2, dninv = pl.pallas_call(
        _tc1_body,
        grid=(nblk,),
        in_specs=[agg4_spec, row_spec, w_spec, w_spec, bias_spec, w_spec,
                  bias_spec],
        out_specs=[half_spec,
                   half_spec,
                   pl.BlockSpec((_BN, 8), lambda i: (i, 0))],
        out_shape=[jax.ShapeDtypeStruct((NP, 64), f32),
                   jax.ShapeDtypeStruct((NP, 64), f32),
                   jax.ShapeDtypeStruct((NP, 8), f32)],
    )(agg1v, x1, wl1, wr1, bl1p, w2cat, b2cat)

    p2c = p2.reshape(NP, 2, 32).transpose(1, 0, 2).reshape(2 * NP, 32)
    agg2c = _make_sc_agg(2)(src_all2, dst_p, p2c, zeros32, zdrain)
    agg2v = agg2c.reshape(2, NP // 4, 128)

    h2 = pl.pallas_call(
        _tc2_body,
        grid=(nblk,),
        in_specs=[agg2_spec, half_spec,
                  pl.BlockSpec((_BN, 8), lambda i: (i, 0))],
        out_specs=half_spec,
        out_shape=jax.ShapeDtypeStruct((NP, 64), f32),
    )(agg2v, s2, dninv)

    x3 = h2[:N].reshape(G, D1)
    x3 = jnp.pad(x3, ((0, GP - G), (0, 0)))
    w1t = jnp.pad(W1.T, ((0, 0), (0, D2P - D2)))
    b1r = jnp.pad(b1, (0, D2P - D2)).reshape(1, D2P)
    hmid = pl.pallas_call(
        _tc3_body,
        grid=(D2P // 256,),
        in_specs=[
            pl.BlockSpec((GP, D1), lambda j: (0, 0)),
            pl.BlockSpec((D1, 256), lambda j: (0, j)),
            pl.BlockSpec((1, 256), lambda j: (0, j)),
        ],
        out_specs=pl.BlockSpec((GP, 256), lambda j: (0, j)),
        out_shape=jax.ShapeDtypeStruct((GP, D2P), f32),
    )(x3, w1t, b1r)

    w2t = jnp.zeros((D2P, 128), f32).at[:D2, :2].set(W2.T)
    b2r = jnp.zeros((1, 128), f32).at[0, :2].set(b2)
    gammap = jnp.pad(gamma, (0, D2P - D2)).reshape(1, D2P)
    betap = jnp.pad(beta, (0, D2P - D2)).reshape(1, D2P)
    out128 = pl.pallas_call(
        _tc4_body,
        in_specs=[
            pl.BlockSpec((GP, D2P), lambda: (0, 0)),
            pl.BlockSpec((1, D2P), lambda: (0, 0)),
            pl.BlockSpec((1, D2P), lambda: (0, 0)),
            pl.BlockSpec((D2P, 128), lambda: (0, 0)),
            pl.BlockSpec((1, 128), lambda: (0, 0)),
        ],
        out_specs=pl.BlockSpec((GP, 128), lambda: (0, 0)),
        out_shape=jax.ShapeDtypeStruct((GP, 128), f32),
    )(hmid, gammap, betap, w2t, b2r)

    return out128[:G, :2]


# speedup vs baseline: 1.1649x; 1.0240x over previous
import functools

import jax
import jax.numpy as jnp
from jax import lax
from jax.experimental import pallas as pl
from jax.experimental.pallas import tpu as pltpu
from jax.experimental.pallas import tpu_sc as plsc

N = 40020
E = 640320
F = 116
H = 64
NP = 40032
B = 128
GB = 5
EPT = 40960
NG = EPT // (GB * B)
E_PAD = 16 * EPT
E_OVER = E_PAD + GB * B
R2 = E_OVER // B
ROWS_PT = EPT // B
RPT = NP // 16
D1 = 116 * 64
D2 = 116 * 32
D2P = 3840
G = 345
GP = 352

_SC_MESH = dict(core_axis_name="c", subcore_axis_name="s",
                num_cores=2, num_subcores=16)


def _softplus(v):
    return jnp.where(v > 20.0, v, jnp.log1p(jnp.exp(jnp.minimum(v, 20.0))))


def _mish(v):
    return v * jnp.tanh(_softplus(v))


@functools.lru_cache(maxsize=None)
def _make_sc_agg(nch):
    cpc = nch // 2

    @functools.partial(
        pl.kernel,
        out_type=jax.ShapeDtypeStruct((nch * NP, 32), jnp.float32),
        mesh=plsc.VectorSubcoreMesh(**_SC_MESH),
        scratch_types=[
            pltpu.VMEM((2, GB, B), jnp.int32),
            pltpu.VMEM((2, GB, B), jnp.int32),
            pltpu.VMEM((2, GB, B), jnp.int32),
            pltpu.VMEM((2, GB, B, 32), jnp.float32),
            pltpu.VMEM_SHARED((NP, 32), jnp.float32),
            pltpu.SemaphoreType.DMA,
            pltpu.SemaphoreType.DMA,
            pltpu.SemaphoreType.DMA,
            pltpu.SemaphoreType.DMA,
            pltpu.SemaphoreType.DMA,
            pltpu.SemaphoreType.DMA,
        ],
        compiler_params=pltpu.CompilerParams(use_tc_tiling_on_sc=False),
    )
    def sc_agg(src_hbm, dst_hbm, table_hbm, zeros_hbm, zdrain_hbm, out_hbm,
               src3, dst3, sdst3, rows, acc,
               gsem0, gsem1, ssem0, ssem1, isem0, isem1):
        c = lax.axis_index("c")
        s = lax.axis_index("s")
        tbl = table_hbm
        row0 = s * RPT
        base_r = s * ROWS_PT
        gsems = (gsem0, gsem1)
        ssems = (ssem0, ssem1)
        isems = (isem0, isem1)

        for ci in range(cpc):
            chunk = c * cpc + ci
            soff = chunk * R2

            def front_half(g, b, goff):
                pltpu.make_async_copy(zdrain_hbm, rows.at[b],
                                      ssems[b]).wait()
                pltpu.make_async_copy(dst_hbm.at[pl.ds(0, GB)],
                                      src3.at[b], isems[b]).wait()
                pltpu.make_async_copy(dst_hbm.at[pl.ds(0, GB)],
                                      dst3.at[b], isems[b]).wait()
                for j in range(GB):
                    for k in range(B // 16):
                        sl = pl.ds(k * 16, 16)
                        sdst3[b, j, sl] = dst3[b, j, sl]
                for j in range(GB):
                    pltpu.async_copy(tbl.at[src3.at[b, j]],
                                     rows.at[b, j], gsems[b])

            def back_half(g, b, goff):
                o = 1 - b
                pltpu.make_async_copy(zdrain_hbm, rows.at[o], gsems[o]).wait()
                for j in range(GB):
                    pltpu.async_copy(rows.at[o, j], acc.at[sdst3.at[o, j]],
                                     ssems[o], add=True)
                pltpu.async_copy(src_hbm.at[pl.ds(soff + goff + GB, GB)],
                                 src3.at[o], isems[o])
                pltpu.async_copy(dst_hbm.at[pl.ds(goff + GB, GB)],
                                 dst3.at[o], isems[o])

            pltpu.sync_copy(zeros_hbm.at[pl.ds(row0, RPT)],
                            acc.at[pl.ds(row0, RPT)])
            plsc.subcore_barrier()

            pltpu.async_copy(zdrain_hbm, rows.at[0], ssems[0])
            pltpu.async_copy(zdrain_hbm, rows.at[1], ssems[1])
            pltpu.async_copy(src_hbm.at[pl.ds(soff + base_r, GB)],
                             src3.at[0], isems[0])
            pltpu.async_copy(dst_hbm.at[pl.ds(base_r, GB)],
                             dst3.at[0], isems[0])
            front_half(0, 0, base_r)
            pltpu.async_copy(src_hbm.at[pl.ds(soff + base_r + GB, GB)],
                             src3.at[1], isems[1])
            pltpu.async_copy(dst_hbm.at[pl.ds(base_r + GB, GB)],
                             dst3.at[1], isems[1])

            def pair(h, carry):
                goff1 = base_r + (2 * h + 1) * GB
                front_half(0, 1, goff1)
                back_half(0, 1, goff1)
                goff0 = goff1 + GB
                front_half(0, 0, goff0)
                back_half(0, 0, goff0)
                return carry

            lax.fori_loop(0, NG // 2 - 1, pair, 0)

            goff_l = base_r + (NG - 1) * GB
            front_half(0, 1, goff_l)
            back_half(0, 1, goff_l)
            pltpu.make_async_copy(zdrain_hbm, rows.at[1], gsems[1]).wait()
            for j in range(GB):
                pltpu.async_copy(rows.at[1, j], acc.at[sdst3.at[1, j]],
                                 ssems[1], add=True)
            pltpu.make_async_copy(zdrain_hbm, rows.at[0], ssems[0]).wait()
            pltpu.make_async_copy(zdrain_hbm, rows.at[1], ssems[1]).wait()
            pltpu.make_async_copy(dst_hbm.at[pl.ds(0, GB)],
                                  src3.at[0], isems[0]).wait()
            pltpu.make_async_copy(dst_hbm.at[pl.ds(0, GB)],
                                  dst3.at[0], isems[0]).wait()

            plsc.subcore_barrier()
            pltpu.sync_copy(acc.at[pl.ds(row0, RPT)],
                            out_hbm.at[pl.ds(chunk * NP + row0, RPT)])

    return sc_agg


_BN = 288


def _unshuffle(v, nch):
    bn4 = v.shape[1]
    rows = []
    for k in range(4):
        rows.append(jnp.concatenate(
            [v[c, :, 32 * k:32 * k + 32] for c in range(nch)], axis=1))
    return jnp.stack(rows, axis=1).reshape(bn4 * 4, nch * 32)


def _tc1_body(agg_ref, x_ref, wl1_ref, wr1_ref, bl1_ref, w2_ref, b2_ref,
              p2_ref, s2_ref, dn_ref):
    a = _unshuffle(agg_ref[...], 4)
    deg = a[:, 116:117]
    dn = 1.0 / jnp.maximum(deg, 1.0)
    acc = jnp.dot(a * dn, wl1_ref[...], preferred_element_type=jnp.float32)
    acc = acc + jnp.dot(x_ref[...], wr1_ref[...],
                        preferred_element_type=jnp.float32)
    h1 = _mish(acc + bl1_ref[...])
    ps = jnp.dot(h1, w2_ref[...],
                 preferred_element_type=jnp.float32) + b2_ref[...]
    bn = ps.shape[0]
    p2_ref[...] = ps[:, :64]
    s2_ref[...] = ps[:, 64:128]
    dn_ref[...] = jnp.broadcast_to(dn, (bn, 8))


def _tc2_body(a2_ref, s2_ref, dn_ref, out_ref):
    a2 = _unshuffle(a2_ref[...], 2)
    dn = dn_ref[:, 0:1]
    out_ref[...] = _mish(a2 * dn + s2_ref[...])


def _tc3_body(x_ref, w_ref, b_ref, out_ref):
    out_ref[...] = lax.dot_general(
        x_ref[...], w_ref[...], (((1,), (1,)), ((), ())),
        preferred_element_type=jnp.float32) + b_ref[...]


def _tc4_body(h_ref, gamma_ref, beta_ref, w_ref, b_ref, out_ref):
    h = h_ref[...]
    mu = jnp.mean(h, axis=-1, keepdims=True)
    var = jnp.mean((h - mu) ** 2, axis=-1, keepdims=True)
    hn = (h - mu) * lax.rsqrt(var + 1e-5)
    hn = hn * gamma_ref[...] + beta_ref[...]
    hm = _mish(hn)
    out_ref[...] = jnp.dot(hm, w_ref[...],
                           preferred_element_type=jnp.float32) + b_ref[...]


def kernel(x, Wl1, bl1, Wr1, Wl2, bl2, Wr2, W1, b1, gamma, beta, W2, b2,
           edge_index):
    f32 = jnp.float32
    src = edge_index[0].astype(jnp.int32)
    dst = edge_index[1].astype(jnp.int32)
    pad_e = jnp.full((E_OVER - E,), N, jnp.int32)
    src_p = jnp.concatenate([src, pad_e]).reshape(R2, B)
    dst_p = jnp.concatenate([dst, pad_e]).reshape(R2, B)
    off4 = (jnp.arange(4, dtype=jnp.int32) * NP)[:, None, None]
    src_all4 = (src_p[None] + off4).reshape(4 * R2, B)
    src_all2 = src_all4[: 2 * R2]
    zdrain = jnp.zeros((GB, B, 32), f32)

    ones_col = jnp.ones((N, 1), f32)
    x1 = jnp.zeros((NP, 128), f32)
    x1 = x1.at[:N, :116].set(x.astype(f32))
    x1 = x1.at[:N, 116:117].set(ones_col)

    zeros32 = jnp.zeros((NP, 32), f32)

    xc = x1.reshape(NP, 4, 32).transpose(1, 0, 2).reshape(4 * NP, 32)
    agg1c = _make_sc_agg(4)(src_all4, dst_p, xc, zeros32, zdrain)
    agg1v = agg1c.reshape(4, NP // 4, 128)

    wl1 = jnp.zeros((128, 128), f32).at[:116, :116].set(Wl1.T)
    wr1 = jnp.zeros((128, 128), f32).at[:116, :116].set(Wr1.T)
    bl1p = jnp.zeros((1, 128), f32).at[0, :116].set(bl1)
    w2cat = jnp.zeros((128, 128), f32)
    w2cat = w2cat.at[:116, :64].set(Wl2.T).at[:116, 64:128].set(Wr2.T)
    b2cat = jnp.zeros((1, 128), f32).at[0, 64:128].set(bl2)

    nblk = NP // _BN
    bn4 = _BN // 4
    row_spec = pl.BlockSpec((_BN, 128), lambda i: (i, 0))
    half_spec = pl.BlockSpec((_BN, 64), lambda i: (i, 0))
    agg4_spec = pl.BlockSpec((4, bn4, 128), lambda i: (0, i, 0))
    agg2_spec = pl.BlockSpec((2, bn4, 128), lambda i: (0, i, 0))
    w_spec = pl.BlockSpec((128, 128), lambda i: (0, 0))
    bias_spec = pl.BlockSpec((1, 128), lambda i: (0, 0))
    p2, s2, dninv = pl.pallas_call(
        _tc1_body,
        grid=(nblk,),
        in_specs=[agg4_spec, row_spec, w_spec, w_spec, bias_spec, w_spec,
                  bias_spec],
        out_specs=[half_spec,
                   half_spec,
                   pl.BlockSpec((_BN, 8), lambda i: (i, 0))],
        out_shape=[jax.ShapeDtypeStruct((NP, 64), f32),
                   jax.ShapeDtypeStruct((NP, 64), f32),
                   jax.ShapeDtypeStruct((NP, 8), f32)],
    )(agg1v, x1, wl1, wr1, bl1p, w2cat, b2cat)

    p2c = p2.reshape(NP, 2, 32).transpose(1, 0, 2).reshape(2 * NP, 32)
    agg2c = _make_sc_agg(2)(src_all2, dst_p, p2c, zeros32, zdrain)
    agg2v = agg2c.reshape(2, NP // 4, 128)

    h2 = pl.pallas_call(
        _tc2_body,
        grid=(nblk,),
        in_specs=[agg2_spec, half_spec,
                  pl.BlockSpec((_BN, 8), lambda i: (i, 0))],
        out_specs=half_spec,
        out_shape=jax.ShapeDtypeStruct((NP, 64), f32),
    )(agg2v, s2, dninv)

    x3 = h2[:N].reshape(G, D1)
    x3 = jnp.pad(x3, ((0, GP - G), (0, 0)))
    b1r = b1.reshape(1, D2)
    hmid = pl.pallas_call(
        _tc3_body,
        grid=(D2 // 128,),
        in_specs=[
            pl.BlockSpec((GP, D1), lambda j: (0, 0)),
            pl.BlockSpec((128, D1), lambda j: (j, 0)),
            pl.BlockSpec((1, 128), lambda j: (0, j)),
        ],
        out_specs=pl.BlockSpec((GP, 128), lambda j: (0, j)),
        out_shape=jax.ShapeDtypeStruct((GP, D2), f32),
    )(x3, W1, b1r)

    w2t = jnp.zeros((D2, 128), f32).at[:, :2].set(W2.T)
    b2r = jnp.zeros((1, 128), f32).at[0, :2].set(b2)
    out128 = pl.pallas_call(
        _tc4_body,
        in_specs=[
            pl.BlockSpec((GP, D2), lambda: (0, 0)),
            pl.BlockSpec((1, D2), lambda: (0, 0)),
            pl.BlockSpec((1, D2), lambda: (0, 0)),
            pl.BlockSpec((D2, 128), lambda: (0, 0)),
            pl.BlockSpec((1, 128), lambda: (0, 0)),
        ],
        out_specs=pl.BlockSpec((GP, 128), lambda: (0, 0)),
        out_shape=jax.ShapeDtypeStruct((GP, 128), f32),
    )(hmid, gamma.reshape(1, D2), beta.reshape(1, D2), w2t, b2r)

    return out128[:G, :2]


# speedup vs baseline: 1.1887x; 1.0205x over previous
import functools

import jax
import jax.numpy as jnp
from jax import lax
from jax.experimental import pallas as pl
from jax.experimental.pallas import tpu as pltpu
from jax.experimental.pallas import tpu_sc as plsc

N = 40020
E = 640320
F = 116
H = 64
NP = 40032
B = 128
GB = 5
EPT = 40960
NG = EPT // (GB * B)
E_PAD = 16 * EPT
E_OVER = E_PAD + GB * B
R2 = E_OVER // B
NGT = R2 // GB
GPT = NG
ROWS_PT = EPT // B
RPT = NP // 16
D1 = 116 * 64
D2 = 116 * 32
D2P = 3840
G = 345
GP = 352

_SC_MESH = dict(core_axis_name="c", subcore_axis_name="s",
                num_cores=2, num_subcores=16)


def _softplus(v):
    return jnp.where(v > 20.0, v, jnp.log1p(jnp.exp(jnp.minimum(v, 20.0))))


def _mish(v):
    return v * jnp.tanh(_softplus(v))


@functools.lru_cache(maxsize=None)
def _make_sc_agg(nch):
    cpc = nch // 2

    @functools.partial(
        pl.kernel,
        out_type=jax.ShapeDtypeStruct((nch * NP, 32), jnp.float32),
        mesh=plsc.VectorSubcoreMesh(**_SC_MESH),
        scratch_types=[
            pltpu.VMEM((2, 2 * GB, B), jnp.int32),
            pltpu.VMEM((2, GB, B), jnp.int32),
            pltpu.VMEM((2, GB, B, 32), jnp.float32),
            pltpu.VMEM_SHARED((NP, 32), jnp.float32),
            pltpu.SemaphoreType.DMA,
            pltpu.SemaphoreType.DMA,
            pltpu.SemaphoreType.DMA,
            pltpu.SemaphoreType.DMA,
            pltpu.SemaphoreType.DMA,
            pltpu.SemaphoreType.DMA,
        ],
        compiler_params=pltpu.CompilerParams(use_tc_tiling_on_sc=False),
    )
    def sc_agg(pack_hbm, table_hbm, zeros_hbm, zdrain_hbm, out_hbm,
               idx3, sdst3, rows, acc,
               gsem0, gsem1, ssem0, ssem1, isem0, isem1):
        c = lax.axis_index("c")
        s = lax.axis_index("s")
        tbl = table_hbm
        row0 = s * RPT
        base_g = s * GPT
        gsems = (gsem0, gsem1)
        ssems = (ssem0, ssem1)
        isems = (isem0, isem1)

        for ci in range(cpc):
            chunk = c * cpc + ci
            soff = chunk * NGT

            def front_half(b):
                pltpu.make_async_copy(zdrain_hbm, rows.at[b],
                                      ssems[b]).wait()
                pltpu.make_async_copy(pack_hbm.at[0],
                                      idx3.at[b], isems[b]).wait()
                for j in range(GB):
                    for k in range(B // 16):
                        sl = pl.ds(k * 16, 16)
                        sdst3[b, j, sl] = idx3[b, GB + j, sl]
                for j in range(GB):
                    pltpu.async_copy(tbl.at[idx3.at[b, j]],
                                     rows.at[b, j], gsems[b])

            def back_half(b, gidx):
                o = 1 - b
                pltpu.make_async_copy(zdrain_hbm, rows.at[o], gsems[o]).wait()
                for j in range(GB):
                    pltpu.async_copy(rows.at[o, j], acc.at[sdst3.at[o, j]],
                                     ssems[o], add=True)
                pltpu.async_copy(pack_hbm.at[soff + gidx + 1],
                                 idx3.at[o], isems[o])

            pltpu.sync_copy(zeros_hbm.at[pl.ds(row0, RPT)],
                            acc.at[pl.ds(row0, RPT)])
            plsc.subcore_barrier()

            pltpu.async_copy(zdrain_hbm, rows.at[0], ssems[0])
            pltpu.async_copy(zdrain_hbm, rows.at[1], ssems[1])
            pltpu.async_copy(pack_hbm.at[soff + base_g],
                             idx3.at[0], isems[0])
            front_half(0)
            pltpu.async_copy(pack_hbm.at[soff + base_g + 1],
                             idx3.at[1], isems[1])

            def pair(h, carry):
                g1 = base_g + 2 * h + 1
                front_half(1)
                back_half(1, g1)
                front_half(0)
                back_half(0, g1 + 1)
                return carry

            lax.fori_loop(0, NG // 2 - 1, pair, 0)

            front_half(1)
            back_half(1, base_g + NG - 1)
            pltpu.make_async_copy(zdrain_hbm, rows.at[1], gsems[1]).wait()
            for j in range(GB):
                pltpu.async_copy(rows.at[1, j], acc.at[sdst3.at[1, j]],
                                 ssems[1], add=True)
            pltpu.make_async_copy(zdrain_hbm, rows.at[0], ssems[0]).wait()
            pltpu.make_async_copy(zdrain_hbm, rows.at[1], ssems[1]).wait()
            pltpu.make_async_copy(pack_hbm.at[0],
                                  idx3.at[0], isems[0]).wait()

            plsc.subcore_barrier()
            pltpu.sync_copy(acc.at[pl.ds(row0, RPT)],
                            out_hbm.at[pl.ds(chunk * NP + row0, RPT)])

    return sc_agg


_BN = 288


def _unshuffle(v, nch):
    bn4 = v.shape[1]
    rows = []
    for k in range(4):
        rows.append(jnp.concatenate(
            [v[c, :, 32 * k:32 * k + 32] for c in range(nch)], axis=1))
    return jnp.stack(rows, axis=1).reshape(bn4 * 4, nch * 32)


def _tc1_body(agg_ref, x_ref, wl1_ref, wr1_ref, bl1_ref, w2_ref, b2_ref,
              p2_ref, s2_ref, dn_ref):
    a = _unshuffle(agg_ref[...], 4)
    deg = a[:, 116:117]
    dn = 1.0 / jnp.maximum(deg, 1.0)
    acc = jnp.dot(a * dn, wl1_ref[...], preferred_element_type=jnp.float32)
    acc = acc + jnp.dot(x_ref[...], wr1_ref[...],
                        preferred_element_type=jnp.float32)
    h1 = _mish(acc + bl1_ref[...])
    ps = jnp.dot(h1, w2_ref[...],
                 preferred_element_type=jnp.float32) + b2_ref[...]
    bn = ps.shape[0]
    p2_ref[...] = ps[:, :64]
    s2_ref[...] = ps[:, 64:128]
    dn_ref[...] = jnp.broadcast_to(dn, (bn, 8))


def _tc2_body(a2_ref, s2_ref, dn_ref, out_ref):
    a2 = _unshuffle(a2_ref[...], 2)
    dn = dn_ref[:, 0:1]
    out_ref[...] = _mish(a2 * dn + s2_ref[...])


def _tc3_body(x_ref, w_ref, b_ref, out_ref):
    out_ref[...] = lax.dot_general(
        x_ref[...], w_ref[...], (((1,), (1,)), ((), ())),
        preferred_element_type=jnp.float32) + b_ref[...]


def _tc4_body(h_ref, gamma_ref, beta_ref, w_ref, b_ref, out_ref):
    h = h_ref[...]
    mu = jnp.mean(h, axis=-1, keepdims=True)
    var = jnp.mean((h - mu) ** 2, axis=-1, keepdims=True)
    hn = (h - mu) * lax.rsqrt(var + 1e-5)
    hn = hn * gamma_ref[...] + beta_ref[...]
    hm = _mish(hn)
    out_ref[...] = jnp.dot(hm, w_ref[...],
                           preferred_element_type=jnp.float32) + b_ref[...]


def kernel(x, Wl1, bl1, Wr1, Wl2, bl2, Wr2, W1, b1, gamma, beta, W2, b2,
           edge_index):
    f32 = jnp.float32
    src = edge_index[0].astype(jnp.int32)
    dst = edge_index[1].astype(jnp.int32)
    pad_e = jnp.full((E_OVER - E,), N, jnp.int32)
    src_p = jnp.concatenate([src, pad_e]).reshape(R2, B)
    dst_p = jnp.concatenate([dst, pad_e]).reshape(R2, B)
    off4 = (jnp.arange(4, dtype=jnp.int32) * NP)[:, None, None, None]
    srcr = src_p.reshape(1, NGT, GB, B)
    dstr = dst_p.reshape(1, NGT, GB, B)
    pack4 = jnp.concatenate(
        [srcr + off4, jnp.broadcast_to(dstr, (4, NGT, GB, B))],
        axis=2).reshape(4 * NGT, 2 * GB, B)
    pack2 = jnp.concatenate(
        [srcr + off4[:2], jnp.broadcast_to(dstr, (2, NGT, GB, B))],
        axis=2).reshape(2 * NGT, 2 * GB, B)
    zdrain = jnp.zeros((GB, B, 32), f32)

    ones_col = jnp.ones((N, 1), f32)
    x1 = jnp.zeros((NP, 128), f32)
    x1 = x1.at[:N, :116].set(x.astype(f32))
    x1 = x1.at[:N, 116:117].set(ones_col)

    zeros32 = jnp.zeros((NP, 32), f32)

    xc = x1.reshape(NP, 4, 32).transpose(1, 0, 2).reshape(4 * NP, 32)
    agg1c = _make_sc_agg(4)(pack4, xc, zeros32, zdrain)
    agg1v = agg1c.reshape(4, NP // 4, 128)

    wl1 = jnp.zeros((128, 128), f32).at[:116, :116].set(Wl1.T)
    wr1 = jnp.zeros((128, 128), f32).at[:116, :116].set(Wr1.T)
    bl1p = jnp.zeros((1, 128), f32).at[0, :116].set(bl1)
    w2cat = jnp.zeros((128, 128), f32)
    w2cat = w2cat.at[:116, :64].set(Wl2.T).at[:116, 64:128].set(Wr2.T)
    b2cat = jnp.zeros((1, 128), f32).at[0, 64:128].set(bl2)

    nblk = NP // _BN
    bn4 = _BN // 4
    row_spec = pl.BlockSpec((_BN, 128), lambda i: (i, 0))
    half_spec = pl.BlockSpec((_BN, 64), lambda i: (i, 0))
    agg4_spec = pl.BlockSpec((4, bn4, 128), lambda i: (0, i, 0))
    agg2_spec = pl.BlockSpec((2, bn4, 128), lambda i: (0, i, 0))
    w_spec = pl.BlockSpec((128, 128), lambda i: (0, 0))
    bias_spec = pl.BlockSpec((1, 128), lambda i: (0, 0))
    p2, s2, dninv = pl.pallas_call(
        _tc1_body,
        grid=(nblk,),
        in_specs=[agg4_spec, row_spec, w_spec, w_spec, bias_spec, w_spec,
                  bias_spec],
        out_specs=[half_spec,
                   half_spec,
                   pl.BlockSpec((_BN, 8), lambda i: (i, 0))],
        out_shape=[jax.ShapeDtypeStruct((NP, 64), f32),
                   jax.ShapeDtypeStruct((NP, 64), f32),
                   jax.ShapeDtypeStruct((NP, 8), f32)],
    )(agg1v, x1, wl1, wr1, bl1p, w2cat, b2cat)

    p2c = p2.reshape(NP, 2, 32).transpose(1, 0, 2).reshape(2 * NP, 32)
    agg2c = _make_sc_agg(2)(pack2, p2c, zeros32, zdrain)
    agg2v = agg2c.reshape(2, NP // 4, 128)

    h2 = pl.pallas_call(
        _tc2_body,
        grid=(nblk,),
        in_specs=[agg2_spec, half_spec,
                  pl.BlockSpec((_BN, 8), lambda i: (i, 0))],
        out_specs=half_spec,
        out_shape=jax.ShapeDtypeStruct((NP, 64), f32),
    )(agg2v, s2, dninv)

    x3 = h2[:N].reshape(G, D1)
    x3 = jnp.pad(x3, ((0, GP - G), (0, 0)))
    b1r = b1.reshape(1, D2)
    hmid = pl.pallas_call(
        _tc3_body,
        grid=(D2 // 128,),
        in_specs=[
            pl.BlockSpec((GP, D1), lambda j: (0, 0)),
            pl.BlockSpec((128, D1), lambda j: (j, 0)),
            pl.BlockSpec((1, 128), lambda j: (0, j)),
        ],
        out_specs=pl.BlockSpec((GP, 128), lambda j: (0, j)),
        out_shape=jax.ShapeDtypeStruct((GP, D2), f32),
    )(x3, W1, b1r)

    w2t = jnp.zeros((D2, 128), f32).at[:, :2].set(W2.T)
    b2r = jnp.zeros((1, 128), f32).at[0, :2].set(b2)
    out128 = pl.pallas_call(
        _tc4_body,
        in_specs=[
            pl.BlockSpec((GP, D2), lambda: (0, 0)),
            pl.BlockSpec((1, D2), lambda: (0, 0)),
            pl.BlockSpec((1, D2), lambda: (0, 0)),
            pl.BlockSpec((D2, 128), lambda: (0, 0)),
            pl.BlockSpec((1, 128), lambda: (0, 0)),
        ],
        out_specs=pl.BlockSpec((GP, 128), lambda: (0, 0)),
        out_shape=jax.ShapeDtypeStruct((GP, 128), f32),
    )(hmid, gamma.reshape(1, D2), beta.reshape(1, D2), w2t, b2r)

    return out128[:G, :2]
